# baseline (speedup 1.0000x reference)
import jax
import jax.numpy as jnp
from jax.experimental import pallas as pl
from jax.experimental.pallas import tpu as pltpu

_SL = 12
_PF = 4
_CPB = 4


def _make_ring_body(N, C, S, inv_s):
    ck = S // _CPB
    nch = N * _CPB

    def src(x_hbm, i):
        return x_hbm.at[i // _CPB, :, pl.ds((i % _CPB) * ck, ck)]

    def dst(o_hbm, i):
        return o_hbm.at[i // _CPB, :, pl.ds((i % _CPB) * ck, ck)]

    def _body(x_hbm, w1_ref, b1_ref, w2_ref, b2_ref, o_hbm,
              ring, sum_scr, max_scr, scale_scr, sin, sout):
        for i in range(min(_PF, nch)):
            pltpu.make_async_copy(
                src(x_hbm, i), ring.at[i % _SL], sin.at[i % _SL]).start()

        for i in range(nch):
            s = i % _SL
            pltpu.make_async_copy(ring.at[s], ring.at[s], sin.at[s]).wait()

            xt = ring[s]
            csum = jnp.sum(xt, axis=-1, keepdims=True)
            cmax = jnp.max(xt, axis=-1, keepdims=True)
            if i % _CPB == 0:
                sum_scr[...] = csum
                max_scr[...] = cmax
            else:
                sum_scr[...] += csum
                max_scr[...] = jnp.maximum(max_scr[...], cmax)

            if i >= _CPB:
                s4 = (i - _CPB) % _SL
                ring[s4] = ring[s4] * scale_scr[...]
                pltpu.make_async_copy(
                    ring.at[s4], dst(o_hbm, i - _CPB), sout.at[s4]).start()

            if i % _CPB == _CPB - 1:
                pstat = jnp.concatenate(
                    [sum_scr[...] * inv_s, max_scr[...]], axis=1)
                h = jnp.dot(w1_ref[...], pstat,
                            preferred_element_type=jnp.float32) + b1_ref[...]
                h = jnp.maximum(h, 0.0)
                a = jnp.dot(w2_ref[...], h,
                            preferred_element_type=jnp.float32) + b2_ref[...]
                scale_scr[...] = jax.nn.sigmoid(a[:, :1] + a[:, 1:2])

            if i + _PF < nch:
                s2 = (i + _PF) % _SL
                if i + _PF >= _SL:
                    pltpu.make_async_copy(
                        ring.at[s2], ring.at[s2], sout.at[s2]).wait()
                pltpu.make_async_copy(
                    src(x_hbm, i + _PF), ring.at[s2], sin.at[s2]).start()

        for j in range(nch - _CPB, nch):
            s4 = j % _SL
            ring[s4] = ring[s4] * scale_scr[...]
            pltpu.make_async_copy(
                ring.at[s4], dst(o_hbm, j), sout.at[s4]).start()
        for j in range(max(0, nch - _SL), nch):
            s4 = j % _SL
            pltpu.make_async_copy(ring.at[s4], ring.at[s4],
                                  sout.at[s4]).wait()

    return _body


def _make_block_kernel(inv_s):
    def _gate_kernel(x_ref, w1_ref, b1_ref, w2_ref, b2_ref, o_ref):
        x = x_ref[0]
        ssum = jnp.sum(x, axis=-1, keepdims=True)
        smax = jnp.max(x, axis=-1, keepdims=True)
        pstat = jnp.concatenate([ssum * inv_s, smax], axis=1)
        h = jnp.dot(w1_ref[...], pstat,
                    preferred_element_type=jnp.float32) + b1_ref[...]
        h = jnp.maximum(h, 0.0)
        a = jnp.dot(w2_ref[...], h,
                    preferred_element_type=jnp.float32) + b2_ref[...]
        scale = jax.nn.sigmoid(a[:, :1] + a[:, 1:2])
        o_ref[0] = (x * scale).astype(o_ref.dtype)

    return _gate_kernel


def kernel(x, w1, b1, w2, b2):
    N, C, D, H, W = x.shape
    S = D * H * W
    Ch = w1.shape[0]

    w1f = jnp.asarray(w1, jnp.float32)
    w2f = jnp.asarray(w2, jnp.float32)
    b1r = jnp.asarray(b1, jnp.float32).reshape(Ch, 1)
    b2r = jnp.asarray(b2, jnp.float32).reshape(C, 1)

    x3 = x.reshape(N, C, S)
    item = jnp.dtype(x.dtype).itemsize

    weight_specs = [
        pl.BlockSpec((Ch, C), lambda *_: (0, 0)),
        pl.BlockSpec((Ch, 1), lambda *_: (0, 0)),
        pl.BlockSpec((C, Ch), lambda *_: (0, 0)),
        pl.BlockSpec((C, 1), lambda *_: (0, 0)),
    ]

    use_ring = (S % (_CPB * 128) == 0 and N * _CPB >= _SL
                and _SL * C * (S // _CPB) * item <= 32 * 1024 * 1024)

    if use_ring:
        ck = S // _CPB
        out3 = pl.pallas_call(
            _make_ring_body(N, C, S, 1.0 / S),
            out_shape=jax.ShapeDtypeStruct((N, C, S), x.dtype),
            grid=(1,),
            in_specs=[pl.BlockSpec(memory_space=pl.ANY)] + weight_specs,
            out_specs=pl.BlockSpec(memory_space=pl.ANY),
            scratch_shapes=[
                pltpu.VMEM((_SL, C, ck), jnp.float32),
                pltpu.VMEM((C, 1), jnp.float32),
                pltpu.VMEM((C, 1), jnp.float32),
                pltpu.VMEM((C, 1), jnp.float32),
                pltpu.SemaphoreType.DMA((_SL,)),
                pltpu.SemaphoreType.DMA((_SL,)),
            ],
            compiler_params=pltpu.CompilerParams(
                dimension_semantics=("arbitrary",),
                vmem_limit_bytes=int(min(
                    _SL * C * ck * item + (8 << 20), 60 * 1024 * 1024)),
            ),
        )(x3, w1f, b1r, w2f, b2r)
    else:
        blk = C * S * item
        out3 = pl.pallas_call(
            _make_block_kernel(1.0 / S),
            out_shape=jax.ShapeDtypeStruct((N, C, S), x.dtype),
            grid=(N,),
            in_specs=[pl.BlockSpec((1, C, S), lambda n: (n, 0, 0))]
            + weight_specs,
            out_specs=pl.BlockSpec((1, C, S), lambda n: (n, 0, 0)),
            compiler_params=pltpu.CompilerParams(
                dimension_semantics=("parallel",),
                vmem_limit_bytes=int(min(4 * blk + (2 << 20),
                                         60 * 1024 * 1024)),
            ),
        )(x3, w1f, b1r, w2f, b2r)
    return out3.reshape(N, C, D, H, W)
